# baseline (speedup 1.0000x reference)
import functools

import jax
import jax.numpy as jnp
from jax import lax
from jax.experimental import pallas as pl
from jax.experimental.pallas import tpu as pltpu
from jax.experimental.pallas import tpu_sc as plsc

B = 16384
EMB = 128
HID = 256
CHUNK = 128


def _sc_gather(users_r, movies_r, user_table, movie_table):
    info = plsc.get_sparse_core_info()
    nc, ns = info.num_cores, info.num_subcores
    nw = nc * ns
    n_rows = users_r.shape[0]
    bs = n_rows * CHUNK
    rows_per_w = n_rows // nw

    mesh = plsc.VectorSubcoreMesh(core_axis_name="c", subcore_axis_name="s")

    n_chunks = 2 * rows_per_w
    n_bufs = min(n_chunks, 6)

    @functools.partial(
        pl.kernel,
        mesh=mesh,
        out_type=(
            jax.ShapeDtypeStruct((bs, EMB), jnp.float32),
            jax.ShapeDtypeStruct((bs, EMB), jnp.float32),
        ),
        scratch_types=[
            pltpu.VMEM((n_chunks, CHUNK), jnp.int32),
            pltpu.VMEM((n_bufs, CHUNK, EMB), jnp.float32),
            pltpu.SemaphoreType.DMA,
            pltpu.SemaphoreType.DMA,
        ],
    )
    def gather_k(u_idx_hbm, m_idx_hbm, utab_hbm, mtab_hbm,
                 u_out_hbm, m_out_hbm, idx_v, bufs, sem_g, sem_o):
        wid = lax.axis_index("s") * nc + lax.axis_index("c")
        base = wid * rows_per_w
        pltpu.sync_copy(u_idx_hbm.at[pl.ds(base, rows_per_w)],
                        idx_v.at[pl.ds(0, rows_per_w)])
        pltpu.sync_copy(m_idx_hbm.at[pl.ds(base, rows_per_w)],
                        idx_v.at[pl.ds(rows_per_w, rows_per_w)])

        def fire(k):
            tab = utab_hbm if k < rows_per_w else mtab_hbm
            return pltpu.async_copy(tab.at[idx_v.at[k]], bufs.at[k % n_bufs],
                                    sem_g)

        def out_copy(k):
            if k < rows_per_w:
                dst = u_out_hbm.at[pl.ds((base + k) * CHUNK, CHUNK)]
            else:
                dst = m_out_hbm.at[pl.ds((base + k - rows_per_w) * CHUNK,
                                         CHUNK)]
            return pltpu.async_copy(bufs.at[k % n_bufs], dst, sem_o)

        gathers = [fire(k) for k in range(n_bufs)]
        outs = []
        for k in range(n_chunks):
            gathers[k].wait()
            outs.append(out_copy(k))
            if k + n_bufs < n_chunks:
                outs[k].wait()
                gathers.append(fire(k + n_bufs))
        for k in range(max(0, n_chunks - n_bufs), n_chunks):
            outs[k].wait()

    return gather_k(users_r, movies_r, user_table, movie_table)


def _mlp_body(u_ref, m_ref, w1_ref, b1_ref, w2_ref, b2_ref, out_ref):
    c16 = jnp.concatenate(
        [u_ref[...].astype(jnp.bfloat16), m_ref[...].astype(jnp.bfloat16)],
        axis=1)
    x = jnp.dot(c16, w1_ref[...], preferred_element_type=jnp.float32) + b1_ref[...]
    x = jnp.maximum(x, 0.0)
    y = lax.dot_general(w2_ref[...], x.astype(jnp.bfloat16),
                        (((1,), (1,)), ((), ())),
                        preferred_element_type=jnp.float32)
    out_ref[...] = y + b2_ref[...]


def _tc_mlp(u_emb, m_emb, W1, b1, W2, b2):
    BK = 1024
    bs = u_emb.shape[0]
    grid = (bs // BK,)
    w1_16 = W1.astype(jnp.bfloat16)
    b1r = b1.reshape(1, HID)
    b2r = b2.reshape(1, 1)
    w2_16 = W2.reshape(1, HID).astype(jnp.bfloat16)
    return pl.pallas_call(
        _mlp_body,
        grid=grid,
        in_specs=[
            pl.BlockSpec((BK, EMB), lambda i: (i, 0)),
            pl.BlockSpec((BK, EMB), lambda i: (i, 0)),
            pl.BlockSpec((2 * EMB, HID), lambda i: (0, 0)),
            pl.BlockSpec((1, HID), lambda i: (0, 0)),
            pl.BlockSpec((1, HID), lambda i: (0, 0)),
            pl.BlockSpec((1, 1), lambda i: (0, 0)),
        ],
        out_specs=pl.BlockSpec((1, BK), lambda i: (0, i)),
        out_shape=jax.ShapeDtypeStruct((1, bs), jnp.float32),
    )(u_emb, m_emb, w1_16, b1r, w2_16, b2r)


def kernel(users, movies, user_table, movie_table, W1, b1, W2, b2):
    users_r = users.reshape(B // CHUNK, CHUNK)
    movies_r = movies.reshape(B // CHUNK, CHUNK)
    u_emb, m_emb = _sc_gather(users_r, movies_r, user_table, movie_table)
    return _tc_mlp(u_emb, m_emb, W1, b1, W2, b2).reshape(B, 1)


# speedup vs baseline: 1.1042x; 1.1042x over previous
import functools

import jax
import jax.numpy as jnp
from jax import lax
from jax.experimental import pallas as pl
from jax.experimental.pallas import tpu as pltpu
from jax.experimental.pallas import tpu_sc as plsc

B = 16384
EMB = 128
HID = 256
CHUNK = 128


def _sc_gather(users_r, movies_r, user_table, movie_table):
    info = plsc.get_sparse_core_info()
    nc, ns = info.num_cores, info.num_subcores
    nw = nc * ns
    n_rows = users_r.shape[0]
    bs = n_rows * CHUNK
    rows_per_w = n_rows // nw

    mesh = plsc.VectorSubcoreMesh(core_axis_name="c", subcore_axis_name="s")

    n_chunks = 2 * rows_per_w
    n_bufs = min(n_chunks, 6)

    @functools.partial(
        pl.kernel,
        mesh=mesh,
        out_type=(
            jax.ShapeDtypeStruct((bs, EMB), jnp.float32),
            jax.ShapeDtypeStruct((bs, EMB), jnp.float32),
        ),
        scratch_types=[
            pltpu.VMEM((n_chunks, CHUNK), jnp.int32),
            pltpu.VMEM((n_bufs, CHUNK, EMB), jnp.float32),
            pltpu.SemaphoreType.DMA,
            pltpu.SemaphoreType.DMA,
        ],
    )
    def gather_k(u_idx_hbm, m_idx_hbm, utab_hbm, mtab_hbm,
                 u_out_hbm, m_out_hbm, idx_v, bufs, sem_g, sem_o):
        wid = lax.axis_index("s") * nc + lax.axis_index("c")
        base = wid * rows_per_w
        pltpu.sync_copy(u_idx_hbm.at[pl.ds(base, rows_per_w)],
                        idx_v.at[pl.ds(0, rows_per_w)])
        pltpu.sync_copy(m_idx_hbm.at[pl.ds(base, rows_per_w)],
                        idx_v.at[pl.ds(rows_per_w, rows_per_w)])

        def fire(k):
            tab = utab_hbm if k < rows_per_w else mtab_hbm
            return pltpu.async_copy(tab.at[idx_v.at[k]], bufs.at[k % n_bufs],
                                    sem_g)

        def out_copy(k):
            if k < rows_per_w:
                dst = u_out_hbm.at[pl.ds((base + k) * CHUNK, CHUNK)]
            else:
                dst = m_out_hbm.at[pl.ds((base + k - rows_per_w) * CHUNK,
                                         CHUNK)]
            return pltpu.async_copy(bufs.at[k % n_bufs], dst, sem_o)

        gathers = [fire(k) for k in range(n_bufs)]
        outs = []
        for k in range(n_chunks):
            gathers[k].wait()
            outs.append(out_copy(k))
            if k + n_bufs < n_chunks:
                outs[k].wait()
                gathers.append(fire(k + n_bufs))
        for k in range(max(0, n_chunks - n_bufs), n_chunks):
            outs[k].wait()

    return gather_k(users_r, movies_r, user_table, movie_table)


def _mlp_body(u_ref, m_ref, w1_ref, b1_ref, w2_ref, b2_ref, out_ref):
    c16 = jnp.concatenate(
        [u_ref[...].astype(jnp.bfloat16), m_ref[...].astype(jnp.bfloat16)],
        axis=1)
    x = jnp.dot(c16, w1_ref[...], preferred_element_type=jnp.float32) + b1_ref[...]
    x = jnp.maximum(x, 0.0)
    y = lax.dot_general(w2_ref[...], x.astype(jnp.bfloat16),
                        (((1,), (1,)), ((), ())),
                        preferred_element_type=jnp.float32)
    out_ref[...] = y + b2_ref[...]


def _tc_mlp(u_emb, m_emb, W1, b1, W2, b2):
    BK = 2048
    bs = u_emb.shape[0]
    grid = (bs // BK,)
    w1_16 = W1.astype(jnp.bfloat16)
    b1r = b1.reshape(1, HID)
    b2r = b2.reshape(1, 1)
    w2_16 = W2.reshape(1, HID).astype(jnp.bfloat16)
    return pl.pallas_call(
        _mlp_body,
        grid=grid,
        in_specs=[
            pl.BlockSpec((BK, EMB), lambda i: (i, 0)),
            pl.BlockSpec((BK, EMB), lambda i: (i, 0)),
            pl.BlockSpec((2 * EMB, HID), lambda i: (0, 0)),
            pl.BlockSpec((1, HID), lambda i: (0, 0)),
            pl.BlockSpec((1, HID), lambda i: (0, 0)),
            pl.BlockSpec((1, 1), lambda i: (0, 0)),
        ],
        out_specs=pl.BlockSpec((1, BK), lambda i: (0, i)),
        out_shape=jax.ShapeDtypeStruct((1, bs), jnp.float32),
    )(u_emb, m_emb, w1_16, b1r, w2_16, b2r)


def kernel(users, movies, user_table, movie_table, W1, b1, W2, b2):
    users_r = users.reshape(B // CHUNK, CHUNK)
    movies_r = movies.reshape(B // CHUNK, CHUNK)
    u_emb, m_emb = _sc_gather(users_r, movies_r, user_table, movie_table)
    return _tc_mlp(u_emb, m_emb, W1, b1, W2, b2).reshape(B, 1)
